# baseline (speedup 1.0000x reference)
import math

import jax
import jax.numpy as jnp
import numpy as np
from jax import lax
from jax.experimental import pallas as pl
from jax.experimental.pallas import tpu as pltpu
from jax.experimental.pallas import tpu_sc as plsc

EMBED = 1024
HALF = EMBED // 2
PAD = 1
BATCH = 4
SEQ = 8192
SCALE = math.log(10000.0) / (HALF - 1)

V_PAD = 8704
ZROWS = 8448
NC, NS, L = 2, 16, 16
NW = NC * NS
FLAT = BATCH * SEQ
SW = SEQ // NW
CH = 16
NCH = SW // CH
NBUF = 4


def _make_table() -> np.ndarray:
    freqs = np.exp(np.arange(HALF, dtype=np.float32) * np.float32(-SCALE))
    pos = np.arange(2, SEQ + 2, dtype=np.float32)
    ang = pos[:, None] * freqs[None, :]
    tab = np.zeros((V_PAD, EMBED), dtype=np.float32)
    tab[:SEQ, :HALF] = np.sin(ang)
    tab[:SEQ, HALF:] = np.cos(ang)
    return tab


_TABLE = _make_table()


def _sc_body(x_hbm, tab_hbm, out_hbm, x_v, buf_v, zbuf_v, zidx_v, *sems):
    gsems, psems = sems[:NBUF], sems[NBUF:NBUF + NBUF]
    fsem = sems[2 * NBUF]
    wid = lax.axis_index("s") * NC + lax.axis_index("c")
    s_base = wid * SW
    for bb in range(BATCH):
        pltpu.sync_copy(x_hbm.at[pl.ds(bb * SEQ + s_base, SW)],
                        x_v.at[bb])
    pltpu.sync_copy(tab_hbm.at[pl.ds(ZROWS, L)], zbuf_v)

    def start_gather(c, b):
        pltpu.async_copy(
            tab_hbm.at[pl.ds(s_base + c * CH, CH)], buf_v.at[b], gsems[b]
        )

    def wait_gather(b):
        pltpu.make_async_copy(
            tab_hbm.at[pl.ds(0, CH)], buf_v.at[b], gsems[b]
        ).wait()

    def start_puts(c, b):
        for bb in range(BATCH):
            pltpu.async_copy(
                buf_v.at[b],
                out_hbm.at[pl.ds(bb * SEQ + s_base + c * CH, CH)],
                psems[b],
            )

    def wait_puts(b):
        for _ in range(BATCH):
            pltpu.make_async_copy(
                buf_v.at[b], out_hbm.at[pl.ds(0, CH)], psems[b]
            ).wait()

    for c in range(NBUF):
        start_gather(c, c)
    for c in range(NCH):
        b = c % NBUF
        nc = c + 2
        if nc >= NBUF:
            bn = nc % NBUF
            wait_puts(bn)
            if nc < NCH:
                start_gather(nc, bn)
        wait_gather(b)
        start_puts(c, b)
    for c in range(NCH - NBUF + 2, NCH):
        wait_puts(c % NBUF)

    iota = lax.iota(jnp.int32, L)
    for bb in range(BATCH):
        out_base = bb * SEQ + s_base

        @pl.loop(0, SW // L)
        def _(v):
            xv = x_v[bb, pl.ds(v * L, L)]
            pm = xv == PAD
            npad = jnp.sum(jnp.where(pm, 1, 0))

            @pl.when(npad > 0)
            def _():
                rows = out_base + v * L + iota
                first = out_base + v * L + plsc.all_reduce_ffs(pm)
                zidx_v[0, :] = jnp.where(pm, rows, first)
                pltpu.async_copy(zbuf_v, out_hbm.at[zidx_v.at[0]], fsem).wait()


def kernel(x):
    tab = jnp.asarray(_TABLE)
    mesh = plsc.VectorSubcoreMesh(core_axis_name="c", subcore_axis_name="s")
    sck = pl.kernel(
        _sc_body,
        out_type=jax.ShapeDtypeStruct((FLAT, EMBED), jnp.float32),
        mesh=mesh,
        scratch_types=[
            pltpu.VMEM((BATCH, SW), jnp.int32),
            pltpu.VMEM((NBUF, CH, EMBED), jnp.float32),
            pltpu.VMEM((L, EMBED), jnp.float32),
            pltpu.VMEM((1, L), jnp.int32),
        ] + [pltpu.SemaphoreType.DMA] * (2 * NBUF + 1),
        compiler_params=pltpu.CompilerParams(needs_layout_passes=False),
    )
    out = sck(x.reshape(FLAT), tab)
    return out.reshape(BATCH, SEQ, EMBED)


# speedup vs baseline: 1.0138x; 1.0138x over previous
import math

import jax
import jax.numpy as jnp
import numpy as np
from jax import lax
from jax.experimental import pallas as pl
from jax.experimental.pallas import tpu as pltpu
from jax.experimental.pallas import tpu_sc as plsc

EMBED = 1024
HALF = EMBED // 2
PAD = 1
BATCH = 4
SEQ = 8192
SCALE = math.log(10000.0) / (HALF - 1)

V_PAD = 8704
ZROWS = 8448
NC, NS, L = 2, 16, 16
NW = NC * NS
FLAT = BATCH * SEQ
SW = SEQ // NW
CH = 32
NCH = SW // CH
NBUF = 3


def _make_table() -> np.ndarray:
    freqs = np.exp(np.arange(HALF, dtype=np.float32) * np.float32(-SCALE))
    pos = np.arange(2, SEQ + 2, dtype=np.float32)
    ang = pos[:, None] * freqs[None, :]
    tab = np.zeros((V_PAD, EMBED), dtype=np.float32)
    tab[:SEQ, :HALF] = np.sin(ang)
    tab[:SEQ, HALF:] = np.cos(ang)
    return tab


_TABLE = _make_table()


def _sc_body(x_hbm, tab_hbm, out_hbm, x_v, buf_v, zbuf_v, zidx_v, *sems):
    gsems, psems = sems[:NBUF], sems[NBUF:NBUF + NBUF]
    fsem = sems[2 * NBUF]
    wid = lax.axis_index("s") * NC + lax.axis_index("c")
    s_base = wid * SW
    for bb in range(BATCH):
        pltpu.sync_copy(x_hbm.at[pl.ds(bb * SEQ + s_base, SW)],
                        x_v.at[bb])
    pltpu.sync_copy(tab_hbm.at[pl.ds(ZROWS, L)], zbuf_v)

    def start_gather(c, b):
        pltpu.async_copy(
            tab_hbm.at[pl.ds(s_base + c * CH, CH)], buf_v.at[b], gsems[b]
        )

    def wait_gather(b):
        pltpu.make_async_copy(
            tab_hbm.at[pl.ds(0, CH)], buf_v.at[b], gsems[b]
        ).wait()

    def start_puts(c, b):
        for bb in range(BATCH):
            pltpu.async_copy(
                buf_v.at[b],
                out_hbm.at[pl.ds(bb * SEQ + s_base + c * CH, CH)],
                psems[b],
            )

    def wait_puts(b):
        for _ in range(BATCH):
            pltpu.make_async_copy(
                buf_v.at[b], out_hbm.at[pl.ds(0, CH)], psems[b]
            ).wait()

    for c in range(NBUF):
        start_gather(c, c)
    for c in range(NCH):
        b = c % NBUF
        nc = c + 2
        if nc >= NBUF:
            bn = nc % NBUF
            wait_puts(bn)
            if nc < NCH:
                start_gather(nc, bn)
        wait_gather(b)
        start_puts(c, b)
    for c in range(NCH - NBUF + 2, NCH):
        wait_puts(c % NBUF)

    iota = lax.iota(jnp.int32, L)
    for bb in range(BATCH):
        out_base = bb * SEQ + s_base

        @pl.loop(0, SW // L)
        def _(v):
            xv = x_v[bb, pl.ds(v * L, L)]
            pm = xv == PAD
            npad = jnp.sum(jnp.where(pm, 1, 0))

            @pl.when(npad > 0)
            def _():
                rows = out_base + v * L + iota
                first = out_base + v * L + plsc.all_reduce_ffs(pm)
                zidx_v[0, :] = jnp.where(pm, rows, first)
                pltpu.async_copy(zbuf_v, out_hbm.at[zidx_v.at[0]], fsem).wait()


def kernel(x):
    tab = jnp.asarray(_TABLE)
    mesh = plsc.VectorSubcoreMesh(core_axis_name="c", subcore_axis_name="s")
    sck = pl.kernel(
        _sc_body,
        out_type=jax.ShapeDtypeStruct((FLAT, EMBED), jnp.float32),
        mesh=mesh,
        scratch_types=[
            pltpu.VMEM((BATCH, SW), jnp.int32),
            pltpu.VMEM((NBUF, CH, EMBED), jnp.float32),
            pltpu.VMEM((L, EMBED), jnp.float32),
            pltpu.VMEM((1, L), jnp.int32),
        ] + [pltpu.SemaphoreType.DMA] * (2 * NBUF + 1),
        compiler_params=pltpu.CompilerParams(needs_layout_passes=False),
    )
    out = sck(x.reshape(FLAT), tab)
    return out.reshape(BATCH, SEQ, EMBED)


# speedup vs baseline: 1.0650x; 1.0505x over previous
import math

import jax
import jax.numpy as jnp
import numpy as np
from jax import lax
from jax.experimental import pallas as pl
from jax.experimental.pallas import tpu as pltpu
from jax.experimental.pallas import tpu_sc as plsc

EMBED = 1024
HALF = EMBED // 2
PAD = 1
BATCH = 4
SEQ = 8192
SCALE = math.log(10000.0) / (HALF - 1)

V_PAD = 8704
ZROWS = 8448
NC, NS, L = 2, 16, 16
NW = NC * NS
FLAT = BATCH * SEQ
SW = SEQ // NW
CH = 32
NCH = SW // CH
NBUF = 3


def _make_table() -> np.ndarray:
    freqs = np.exp(np.arange(HALF, dtype=np.float32) * np.float32(-SCALE))
    pos = np.arange(2, SEQ + 2, dtype=np.float32)
    ang = pos[:, None] * freqs[None, :]
    tab = np.zeros((V_PAD, EMBED), dtype=np.float32)
    tab[:SEQ, :HALF] = np.sin(ang)
    tab[:SEQ, HALF:] = np.cos(ang)
    return tab


_TABLE = _make_table()


def _sc_body(x_hbm, tab_hbm, out_hbm, x_v, buf_v, zbuf_v, zidx_v, *sems):
    gsems, psems = sems[:NBUF], sems[NBUF:NBUF + NBUF]
    fsem, xsem = sems[2 * NBUF], sems[2 * NBUF + 1]
    wid = lax.axis_index("s") * NC + lax.axis_index("c")
    s_base = wid * SW

    def start_gather(c, b):
        pltpu.async_copy(
            tab_hbm.at[pl.ds(s_base + c * CH, CH)], buf_v.at[b], gsems[b]
        )

    def wait_gather(b):
        pltpu.make_async_copy(
            tab_hbm.at[pl.ds(0, CH)], buf_v.at[b], gsems[b]
        ).wait()

    def start_puts(c, b):
        for bb in range(BATCH):
            pltpu.async_copy(
                buf_v.at[b],
                out_hbm.at[pl.ds(bb * SEQ + s_base + c * CH, CH)],
                psems[b],
            )

    def wait_puts(b):
        for _ in range(BATCH):
            pltpu.make_async_copy(
                buf_v.at[b], out_hbm.at[pl.ds(0, CH)], psems[b]
            ).wait()

    for c in range(NBUF):
        start_gather(c, c)
    for bb in range(BATCH):
        pltpu.async_copy(x_hbm.at[pl.ds(bb * SEQ + s_base, SW)],
                         x_v.at[bb], xsem)
    pltpu.async_copy(tab_hbm.at[pl.ds(ZROWS, L)], zbuf_v, xsem)
    for c in range(NCH):
        b = c % NBUF
        nc = c + 2
        if nc >= NBUF:
            bn = nc % NBUF
            wait_puts(bn)
            if nc < NCH:
                start_gather(nc, bn)
        wait_gather(b)
        start_puts(c, b)
    for c in range(NCH - NBUF + 2, NCH):
        wait_puts(c % NBUF)

    for bb in range(BATCH):
        pltpu.make_async_copy(x_hbm.at[pl.ds(0, SW)], x_v.at[bb], xsem).wait()
    pltpu.make_async_copy(tab_hbm.at[pl.ds(ZROWS, L)], zbuf_v, xsem).wait()

    iota = lax.iota(jnp.int32, L)
    for bb in range(BATCH):
        out_base = bb * SEQ + s_base

        @pl.loop(0, SW // L)
        def _(v):
            xv = x_v[bb, pl.ds(v * L, L)]
            pm = xv == PAD
            npad = jnp.sum(jnp.where(pm, 1, 0))

            @pl.when(npad > 0)
            def _():
                rows = out_base + v * L + iota
                first = out_base + v * L + plsc.all_reduce_ffs(pm)
                zidx_v[0, :] = jnp.where(pm, rows, first)
                pltpu.async_copy(zbuf_v, out_hbm.at[zidx_v.at[0]], fsem).wait()


def kernel(x):
    tab = jnp.asarray(_TABLE)
    mesh = plsc.VectorSubcoreMesh(core_axis_name="c", subcore_axis_name="s")
    sck = pl.kernel(
        _sc_body,
        out_type=jax.ShapeDtypeStruct((FLAT, EMBED), jnp.float32),
        mesh=mesh,
        scratch_types=[
            pltpu.VMEM((BATCH, SW), jnp.int32),
            pltpu.VMEM((NBUF, CH, EMBED), jnp.float32),
            pltpu.VMEM((L, EMBED), jnp.float32),
            pltpu.VMEM((1, L), jnp.int32),
        ] + [pltpu.SemaphoreType.DMA] * (2 * NBUF + 2),
        compiler_params=pltpu.CompilerParams(needs_layout_passes=False),
    )
    out = sck(x.reshape(FLAT), tab)
    return out.reshape(BATCH, SEQ, EMBED)


# speedup vs baseline: 1.0842x; 1.0180x over previous
import math

import jax
import jax.numpy as jnp
import numpy as np
from jax import lax
from jax.experimental import pallas as pl
from jax.experimental.pallas import tpu as pltpu
from jax.experimental.pallas import tpu_sc as plsc

EMBED = 1024
HALF = EMBED // 2
PAD = 1
BATCH = 4
SEQ = 8192
SCALE = math.log(10000.0) / (HALF - 1)

V_PAD = 8208
ZROWS = 8192
NC, NS, L = 2, 16, 16
NW = NC * NS
FLAT = BATCH * SEQ
SW = SEQ // NW
CH = 32
NCH = SW // CH
NBUF = 3


def _make_table() -> np.ndarray:
    freqs = np.exp(np.arange(HALF, dtype=np.float32) * np.float32(-SCALE))
    pos = np.arange(2, SEQ + 2, dtype=np.float32)
    ang = pos[:, None] * freqs[None, :]
    tab = np.zeros((V_PAD, EMBED), dtype=np.float32)
    tab[:SEQ, :HALF] = np.sin(ang)
    tab[:SEQ, HALF:] = np.cos(ang)
    return tab


_TABLE = _make_table()


def _sc_body(x_hbm, tab_hbm, out_hbm, x_v, buf_v, zbuf_v, zidx_v, *sems):
    gsems, psems = sems[:NBUF], sems[NBUF:NBUF + NBUF]
    fsem, xsem = sems[2 * NBUF], sems[2 * NBUF + 1]
    wid = lax.axis_index("s") * NC + lax.axis_index("c")
    s_base = wid * SW

    def start_gather(c, b):
        pltpu.async_copy(
            tab_hbm.at[pl.ds(s_base + c * CH, CH)], buf_v.at[b], gsems[b]
        )

    def wait_gather(b):
        pltpu.make_async_copy(
            tab_hbm.at[pl.ds(0, CH)], buf_v.at[b], gsems[b]
        ).wait()

    def start_puts(c, b):
        for bb in range(BATCH):
            pltpu.async_copy(
                buf_v.at[b],
                out_hbm.at[pl.ds(bb * SEQ + s_base + c * CH, CH)],
                psems[b],
            )

    def wait_puts(b):
        for _ in range(BATCH):
            pltpu.make_async_copy(
                buf_v.at[b], out_hbm.at[pl.ds(0, CH)], psems[b]
            ).wait()

    for c in range(NBUF):
        start_gather(c, c)
    for bb in range(BATCH):
        pltpu.async_copy(x_hbm.at[pl.ds(bb * SEQ + s_base, SW)],
                         x_v.at[bb], xsem)
    pltpu.async_copy(tab_hbm.at[pl.ds(ZROWS, L)], zbuf_v, xsem)
    for c in range(NCH):
        b = c % NBUF
        wait_gather(b)
        start_puts(c, b)
        nc = c + 2
        if NBUF <= nc < NCH:
            bn = nc % NBUF
            wait_puts(bn)
            start_gather(nc, bn)
    for c in range(NCH - NBUF, NCH):
        wait_puts(c % NBUF)

    for bb in range(BATCH):
        pltpu.make_async_copy(x_hbm.at[pl.ds(0, SW)], x_v.at[bb], xsem).wait()
    pltpu.make_async_copy(tab_hbm.at[pl.ds(ZROWS, L)], zbuf_v, xsem).wait()

    iota = lax.iota(jnp.int32, L)
    for bb in range(BATCH):
        out_base = bb * SEQ + s_base

        @pl.loop(0, SW // L)
        def _(v):
            xv = x_v[bb, pl.ds(v * L, L)]
            pm = xv == PAD
            npad = jnp.sum(jnp.where(pm, 1, 0))

            @pl.when(npad > 0)
            def _():
                rows = out_base + v * L + iota
                first = out_base + v * L + plsc.all_reduce_ffs(pm)
                zidx_v[0, :] = jnp.where(pm, rows, first)
                pltpu.async_copy(zbuf_v, out_hbm.at[zidx_v.at[0]], fsem).wait()


def kernel(x):
    tab = jnp.asarray(_TABLE)
    mesh = plsc.VectorSubcoreMesh(core_axis_name="c", subcore_axis_name="s")
    sck = pl.kernel(
        _sc_body,
        out_type=jax.ShapeDtypeStruct((FLAT, EMBED), jnp.float32),
        mesh=mesh,
        scratch_types=[
            pltpu.VMEM((BATCH, SW), jnp.int32),
            pltpu.VMEM((NBUF, CH, EMBED), jnp.float32),
            pltpu.VMEM((L, EMBED), jnp.float32),
            pltpu.VMEM((1, L), jnp.int32),
        ] + [pltpu.SemaphoreType.DMA] * (2 * NBUF + 2),
        compiler_params=pltpu.CompilerParams(needs_layout_passes=False),
    )
    out = sck(x.reshape(FLAT), tab)
    return out.reshape(BATCH, SEQ, EMBED)
